# baseline (speedup 1.0000x reference)
import functools

import jax
import jax.numpy as jnp
from jax import lax
from jax.experimental import pallas as pl
from jax.experimental.pallas import tpu as pltpu
from jax.experimental.pallas import tpu_sc as plsc


@functools.lru_cache(maxsize=None)
def _make_gather_kernel(V, D, B):
    info = plsc.get_sparse_core_info()
    num_cores, num_subcores = info.num_cores, info.num_subcores
    num_workers = num_cores * num_subcores
    assert B % num_workers == 0
    b_per_w = B // num_workers

    n_chunks = 8
    rows_per_chunk = b_per_w // n_chunks

    mesh = plsc.VectorSubcoreMesh(core_axis_name="c", subcore_axis_name="s")

    @functools.partial(
        pl.kernel,
        mesh=mesh,
        out_type=jax.ShapeDtypeStruct((B, D), jnp.float32),
        scratch_types=[
            pltpu.VMEM((b_per_w,), jnp.int32),
            pltpu.VMEM((b_per_w, D), jnp.float32),
            [pltpu.SemaphoreType.DMA] * n_chunks,
            pltpu.SemaphoreType.DMA,
        ],
    )
    def gather_kernel(idx_hbm, table_hbm, out_hbm, idx_v, rows_v, gsems, osem):
        wid = lax.axis_index("s") * num_cores + lax.axis_index("c")
        base = wid * b_per_w
        pltpu.sync_copy(idx_hbm.at[pl.ds(base, b_per_w)], idx_v)
        gathers = []
        for c in range(n_chunks):
            lo = c * rows_per_chunk
            gathers.append(
                pltpu.async_copy(
                    table_hbm.at[idx_v.at[pl.ds(lo, rows_per_chunk)]],
                    rows_v.at[pl.ds(lo, rows_per_chunk)],
                    gsems[c],
                )
            )
        writes = []
        for c in range(n_chunks):
            gathers[c].wait()
            lo = c * rows_per_chunk
            writes.append(
                pltpu.async_copy(
                    rows_v.at[pl.ds(lo, rows_per_chunk)],
                    out_hbm.at[pl.ds(base + lo, rows_per_chunk)],
                    osem,
                )
            )
        for w in writes:
            w.wait()

    return gather_kernel


@jax.jit
def kernel(inputs, weights):
    (B,) = inputs.shape
    V, D = weights.shape
    gather = _make_gather_kernel(V, D, B)
    return gather(inputs, weights)


# speedup vs baseline: 1.0141x; 1.0141x over previous
import functools

import jax
import jax.numpy as jnp
from jax import lax
from jax.experimental import pallas as pl
from jax.experimental.pallas import tpu as pltpu
from jax.experimental.pallas import tpu_sc as plsc


@functools.lru_cache(maxsize=None)
def _make_gather_kernel(V, D, B):
    info = plsc.get_sparse_core_info()
    num_cores, num_subcores = 1, info.num_subcores
    num_workers = num_cores * num_subcores
    assert B % num_workers == 0
    b_per_w = B // num_workers

    n_chunks = 4
    rows_per_chunk = b_per_w // n_chunks

    mesh = plsc.VectorSubcoreMesh(
        core_axis_name="c", subcore_axis_name="s", num_cores=num_cores
    )

    @functools.partial(
        pl.kernel,
        mesh=mesh,
        out_type=jax.ShapeDtypeStruct((B, D), jnp.float32),
        scratch_types=[
            pltpu.VMEM((b_per_w,), jnp.int32),
            pltpu.VMEM((b_per_w, D), jnp.float32),
            [pltpu.SemaphoreType.DMA] * n_chunks,
            pltpu.SemaphoreType.DMA,
        ],
    )
    def gather_kernel(idx_hbm, table_hbm, out_hbm, idx_v, rows_v, gsems, osem):
        wid = lax.axis_index("s") * num_cores + lax.axis_index("c")
        base = wid * b_per_w
        pltpu.sync_copy(idx_hbm.at[pl.ds(base, b_per_w)], idx_v)
        gathers = []
        for c in range(n_chunks):
            lo = c * rows_per_chunk
            gathers.append(
                pltpu.async_copy(
                    table_hbm.at[idx_v.at[pl.ds(lo, rows_per_chunk)]],
                    rows_v.at[pl.ds(lo, rows_per_chunk)],
                    gsems[c],
                )
            )
        writes = []
        for c in range(n_chunks):
            gathers[c].wait()
            lo = c * rows_per_chunk
            writes.append(
                pltpu.async_copy(
                    rows_v.at[pl.ds(lo, rows_per_chunk)],
                    out_hbm.at[pl.ds(base + lo, rows_per_chunk)],
                    osem,
                )
            )
        for w in writes:
            w.wait()

    return gather_kernel


@jax.jit
def kernel(inputs, weights):
    (B,) = inputs.shape
    V, D = weights.shape
    gather = _make_gather_kernel(V, D, B)
    return gather(inputs, weights)


# speedup vs baseline: 1.0305x; 1.0162x over previous
import functools

import jax
import jax.numpy as jnp
from jax import lax
from jax.experimental import pallas as pl
from jax.experimental.pallas import tpu as pltpu
from jax.experimental.pallas import tpu_sc as plsc


@functools.lru_cache(maxsize=None)
def _make_gather_kernel(V, D, B):
    info = plsc.get_sparse_core_info()
    num_cores, num_subcores = 1, info.num_subcores
    num_workers = num_cores * num_subcores
    assert B % num_workers == 0
    b_per_w = B // num_workers

    n_chunks = 8
    rows_per_chunk = b_per_w // n_chunks

    mesh = plsc.VectorSubcoreMesh(
        core_axis_name="c", subcore_axis_name="s", num_cores=num_cores
    )

    @functools.partial(
        pl.kernel,
        mesh=mesh,
        out_type=jax.ShapeDtypeStruct((B, D), jnp.float32),
        scratch_types=[
            pltpu.VMEM((b_per_w,), jnp.int32),
            pltpu.VMEM((b_per_w, D), jnp.float32),
            pltpu.SemaphoreType.DMA,
            [pltpu.SemaphoreType.DMA] * n_chunks,
            pltpu.SemaphoreType.DMA,
        ],
    )
    def gather_kernel(idx_hbm, table_hbm, out_hbm, idx_v, rows_v, isem, gsems, osem):
        wid = lax.axis_index("s") * num_cores + lax.axis_index("c")
        base = wid * b_per_w
        idx_loads = []
        for c in range(n_chunks):
            lo = c * rows_per_chunk
            idx_loads.append(
                pltpu.async_copy(
                    idx_hbm.at[pl.ds(base + lo, rows_per_chunk)],
                    idx_v.at[pl.ds(lo, rows_per_chunk)],
                    isem,
                )
            )
        gathers = []
        for c in range(n_chunks):
            idx_loads[c].wait()
            lo = c * rows_per_chunk
            gathers.append(
                pltpu.async_copy(
                    table_hbm.at[idx_v.at[pl.ds(lo, rows_per_chunk)]],
                    rows_v.at[pl.ds(lo, rows_per_chunk)],
                    gsems[c],
                )
            )
        writes = []
        for c in range(n_chunks):
            gathers[c].wait()
            lo = c * rows_per_chunk
            writes.append(
                pltpu.async_copy(
                    rows_v.at[pl.ds(lo, rows_per_chunk)],
                    out_hbm.at[pl.ds(base + lo, rows_per_chunk)],
                    osem,
                )
            )
        for w in writes:
            w.wait()

    return gather_kernel


@jax.jit
def kernel(inputs, weights):
    (B,) = inputs.shape
    V, D = weights.shape
    gather = _make_gather_kernel(V, D, B)
    return gather(inputs, weights)
